# baseline (speedup 1.0000x reference)
import jax
import jax.numpy as jnp
from jax.experimental import pallas as pl
from jax.experimental.pallas import tpu as pltpu

_TILE_N = 65536


def _cdiv(a, b):
    return (a + b - 1) // b


def _sinpi(a):
    m = jnp.round(a)
    u = a - m
    u2 = u * u
    p = 0.07788842755804198
    p = p * u2 - 0.5983952285608748
    p = p * u2 + 2.5500918969050588
    p = p * u2 - 5.1677107041503625
    p = p * u2 + 3.1415926441702
    su = u * p
    sb = (m.astype(jnp.int32) & 1) << 31
    return jax.lax.bitcast_convert_type(
        jax.lax.bitcast_convert_type(su, jnp.int32) ^ sb, jnp.float32)


def kernel(x, W0, b0, W1, b1, W2, b2, W3, b3, W4, b4, W5, b5):
    f32 = jnp.float32
    N, d_in = x.shape
    Dh = W0.shape[0]
    Dp = W4.shape[0]
    d_out = W5.shape[0]

    tile_n = _TILE_N
    num_tiles = _cdiv(N, tile_n)
    inv_n = 1.0 / N

    x = x.astype(f32)
    xT = x.T

    ip = 1.0 / jnp.pi
    W0c = W0.astype(f32) * ip
    b0c = b0.astype(f32).reshape(Dh, 1) * ip
    W1c = W1.astype(f32) * ip
    b1c = b1.astype(f32).reshape(Dh, 1) * ip
    W2c = W2.astype(f32) * ip
    W2x = W2c[:, :d_in]
    b2c = b2.astype(f32).reshape(Dh, 1) * ip
    W3c = W3.astype(f32) * ip
    b3c = b3.astype(f32).reshape(Dh, 1) * ip
    W4c = W4.astype(f32)
    W4a = W4c[:, :Dh] * ip
    W4ax = W4c[:, :d_in] * ip
    W4b = W4c[:, Dh:] * ip
    b4c = b4.astype(f32).reshape(Dp, 1) * ip
    W5c = W5.astype(f32)
    W5x = W5c[:, :d_in]
    b5c = b5.astype(f32).reshape(d_out, 1)

    vmem_limit = 48 * 1024 * 1024

    def pass0_kernel(x_ref, W0_ref, b0_ref, W1_ref, b1_ref, W2_ref, W2x_ref,
                     b2_ref, W3_ref, b3_ref, psum_ref):
        t = pl.program_id(0)
        xv = x_ref[...]

        h = _sinpi(jnp.dot(W0_ref[...], xv, preferred_element_type=f32)
                    + b0_ref[...])
        s1 = _sinpi(jnp.dot(W1_ref[...], h, preferred_element_type=f32)
                     + b1_ref[...])
        u = _sinpi(jnp.dot(W2_ref[...], s1, preferred_element_type=f32)
                    + jnp.dot(W2x_ref[...], xv, preferred_element_type=f32)
                    + b2_ref[...])
        v = _sinpi(jnp.dot(W3_ref[...], u, preferred_element_type=f32)
                    + b3_ref[...])
        g = v + s1

        def emit(gv, xvv):
            ps = jnp.sum(gv, axis=1, keepdims=True)
            px = jnp.sum(xvv, axis=1, keepdims=True)
            psum_ref[...] = ps
            psum_ref[0:d_in, :] = ps[0:d_in, :] + px

        last_ragged = (t + 1) * tile_n > N

        @pl.when(jnp.logical_not(last_ragged))
        def _():
            emit(g, xv)

        @pl.when(last_ragged)
        def _():
            lane = jax.lax.broadcasted_iota(jnp.int32, (1, tile_n), 1)
            valid = (lane + t * tile_n) < N
            emit(jnp.where(valid, g, 0.0), jnp.where(valid, xv, 0.0))

    psum = pl.pallas_call(
        pass0_kernel,
        out_shape=jax.ShapeDtypeStruct((num_tiles, Dh, 1), f32),
        grid_spec=pltpu.PrefetchScalarGridSpec(
            num_scalar_prefetch=0,
            grid=(num_tiles,),
            in_specs=[
                pl.BlockSpec((d_in, tile_n), lambda t: (0, t)),
                pl.BlockSpec((Dh, d_in), lambda t: (0, 0)),
                pl.BlockSpec((Dh, 1), lambda t: (0, 0)),
                pl.BlockSpec((Dh, Dh), lambda t: (0, 0)),
                pl.BlockSpec((Dh, 1), lambda t: (0, 0)),
                pl.BlockSpec((Dh, Dh), lambda t: (0, 0)),
                pl.BlockSpec((Dh, d_in), lambda t: (0, 0)),
                pl.BlockSpec((Dh, 1), lambda t: (0, 0)),
                pl.BlockSpec((Dh, Dh), lambda t: (0, 0)),
                pl.BlockSpec((Dh, 1), lambda t: (0, 0)),
            ],
            out_specs=pl.BlockSpec((None, Dh, 1), lambda t: (t, 0, 0)),
        ),
        compiler_params=pltpu.CompilerParams(
            dimension_semantics=("parallel",),
            vmem_limit_bytes=vmem_limit),
        cost_estimate=pl.CostEstimate(
            flops=int(N * (2 * Dh * d_in * 2 + 3 * 2 * Dh * Dh + 2 * Dh)),
            transcendentals=int(N * 4 * Dh),
            bytes_accessed=int(4 * (d_in * N + Dh * num_tiles))),
    )(xT, W0c, b0c, W1c, b1c, W2c, W2x, b2c, W3c, b3c)

    mean = (jnp.sum(psum[:, :, 0], axis=0) * inv_n).reshape(Dh, 1)
    c4 = jnp.dot(W4b, mean) + b4c

    def pass1_kernel(x_ref, W0_ref, b0_ref, W1_ref, b1_ref,
                     W4a_ref, W4ax_ref, c4_ref, W5_ref, W5x_ref,
                     b5_ref, out_ref):
        xv = x_ref[...]
        h = _sinpi(jnp.dot(W0_ref[...], xv, preferred_element_type=f32)
                    + b0_ref[...])
        s1 = _sinpi(jnp.dot(W1_ref[...], h, preferred_element_type=f32)
                     + b1_ref[...])
        s = _sinpi(jnp.dot(W4a_ref[...], s1, preferred_element_type=f32)
                    + jnp.dot(W4ax_ref[...], xv, preferred_element_type=f32)
                    + c4_ref[...])
        out_ref[...] = (jnp.dot(W5_ref[...], s, preferred_element_type=f32)
                        + jnp.dot(W5x_ref[...], xv, preferred_element_type=f32)
                        + b5_ref[...])

    out = pl.pallas_call(
        pass1_kernel,
        out_shape=jax.ShapeDtypeStruct((d_out, N), f32),
        grid_spec=pltpu.PrefetchScalarGridSpec(
            num_scalar_prefetch=0,
            grid=(num_tiles,),
            in_specs=[
                pl.BlockSpec((d_in, tile_n), lambda t: (0, t)),
                pl.BlockSpec((Dh, d_in), lambda t: (0, 0)),
                pl.BlockSpec((Dh, 1), lambda t: (0, 0)),
                pl.BlockSpec((Dh, Dh), lambda t: (0, 0)),
                pl.BlockSpec((Dh, 1), lambda t: (0, 0)),
                pl.BlockSpec((Dp, Dh), lambda t: (0, 0)),
                pl.BlockSpec((Dp, d_in), lambda t: (0, 0)),
                pl.BlockSpec((Dp, 1), lambda t: (0, 0)),
                pl.BlockSpec((d_out, Dp), lambda t: (0, 0)),
                pl.BlockSpec((d_out, d_in), lambda t: (0, 0)),
                pl.BlockSpec((d_out, 1), lambda t: (0, 0)),
            ],
            out_specs=pl.BlockSpec((d_out, tile_n), lambda t: (0, t)),
        ),
        compiler_params=pltpu.CompilerParams(
            dimension_semantics=("parallel",),
            vmem_limit_bytes=vmem_limit),
        cost_estimate=pl.CostEstimate(
            flops=int(N * (2 * Dh * d_in * 2 + 2 * Dh * Dh + 2 * Dp * Dh
                           + 2 * d_out * Dp)),
            transcendentals=int(N * (2 * Dh + Dp)),
            bytes_accessed=int(4 * ((d_in + d_out) * N + Dh * num_tiles))),
    )(xT, W0c, b0c, W1c, b1c, W4a, W4ax, c4, W5c, W5x, b5c)

    return out.T


# speedup vs baseline: 1.2115x; 1.2115x over previous
import jax
import jax.numpy as jnp
from jax.experimental import pallas as pl
from jax.experimental.pallas import tpu as pltpu

_TILE_N = 32768


def _cdiv(a, b):
    return (a + b - 1) // b


def _sinpi(a):
    m = jnp.round(a)
    u = a - m
    u2 = u * u
    p = 0.07788842755804198
    p = p * u2 - 0.5983952285608748
    p = p * u2 + 2.5500918969050588
    p = p * u2 - 5.1677107041503625
    p = p * u2 + 3.1415926441702
    su = u * p
    sb = (m.astype(jnp.int32) & 1) << 31
    return jax.lax.bitcast_convert_type(
        jax.lax.bitcast_convert_type(su, jnp.int32) ^ sb, jnp.float32)


def kernel(x, W0, b0, W1, b1, W2, b2, W3, b3, W4, b4, W5, b5):
    f32 = jnp.float32
    N, d_in = x.shape
    Dh = W0.shape[0]
    Dp = W4.shape[0]
    d_out = W5.shape[0]

    tile_n = _TILE_N
    num_tiles = _cdiv(N, tile_n)
    inv_n = 1.0 / N

    x = x.astype(f32)
    xT = x.T

    ip = 1.0 / jnp.pi
    W0c = W0.astype(f32) * ip
    b0c = b0.astype(f32).reshape(Dh, 1) * ip
    W1c = W1.astype(f32) * ip
    b1c = b1.astype(f32).reshape(Dh, 1) * ip
    W2c = W2.astype(f32) * ip
    W2x = W2c[:, :d_in]
    b2c = b2.astype(f32).reshape(Dh, 1) * ip
    W3c = W3.astype(f32) * ip
    b3c = b3.astype(f32).reshape(Dh, 1) * ip
    W4c = W4.astype(f32)
    W4a = W4c[:, :Dh] * ip
    W4ax = W4c[:, :d_in] * ip
    W4b = W4c[:, Dh:] * ip
    b4c = b4.astype(f32).reshape(Dp, 1) * ip
    W5c = W5.astype(f32)
    W5x = W5c[:, :d_in]
    b5c = b5.astype(f32).reshape(d_out, 1)

    vmem_limit = 48 * 1024 * 1024

    def pass0_kernel(x_ref, W0_ref, b0_ref, W1_ref, b1_ref, W2_ref, W2x_ref,
                     b2_ref, W3_ref, b3_ref, psum_ref, s1c_ref):
        t = pl.program_id(0)
        xv = x_ref[...]

        h = _sinpi(jnp.dot(W0_ref[...], xv, preferred_element_type=f32)
                    + b0_ref[...])
        s1 = _sinpi(jnp.dot(W1_ref[...], h, preferred_element_type=f32)
                     + b1_ref[...])
        s1c_ref[...] = s1.astype(jnp.bfloat16)
        u = _sinpi(jnp.dot(W2_ref[...], s1, preferred_element_type=f32)
                    + jnp.dot(W2x_ref[...], xv, preferred_element_type=f32)
                    + b2_ref[...])
        v = _sinpi(jnp.dot(W3_ref[...], u, preferred_element_type=f32)
                    + b3_ref[...])
        g = v + s1

        def emit(gv, xvv):
            ps = jnp.sum(gv, axis=1, keepdims=True)
            px = jnp.sum(xvv, axis=1, keepdims=True)
            psum_ref[...] = ps
            psum_ref[0:d_in, :] = ps[0:d_in, :] + px

        last_ragged = (t + 1) * tile_n > N

        @pl.when(jnp.logical_not(last_ragged))
        def _():
            emit(g, xv)

        @pl.when(last_ragged)
        def _():
            lane = jax.lax.broadcasted_iota(jnp.int32, (1, tile_n), 1)
            valid = (lane + t * tile_n) < N
            emit(jnp.where(valid, g, 0.0), jnp.where(valid, xv, 0.0))

    psum, s1c = pl.pallas_call(
        pass0_kernel,
        out_shape=(jax.ShapeDtypeStruct((num_tiles, Dh, 1), f32),
                   jax.ShapeDtypeStruct((Dh, N), jnp.bfloat16)),
        grid_spec=pltpu.PrefetchScalarGridSpec(
            num_scalar_prefetch=0,
            grid=(num_tiles,),
            in_specs=[
                pl.BlockSpec((d_in, tile_n), lambda t: (0, t)),
                pl.BlockSpec((Dh, d_in), lambda t: (0, 0)),
                pl.BlockSpec((Dh, 1), lambda t: (0, 0)),
                pl.BlockSpec((Dh, Dh), lambda t: (0, 0)),
                pl.BlockSpec((Dh, 1), lambda t: (0, 0)),
                pl.BlockSpec((Dh, Dh), lambda t: (0, 0)),
                pl.BlockSpec((Dh, d_in), lambda t: (0, 0)),
                pl.BlockSpec((Dh, 1), lambda t: (0, 0)),
                pl.BlockSpec((Dh, Dh), lambda t: (0, 0)),
                pl.BlockSpec((Dh, 1), lambda t: (0, 0)),
            ],
            out_specs=(pl.BlockSpec((None, Dh, 1), lambda t: (t, 0, 0)),
                       pl.BlockSpec((Dh, tile_n), lambda t: (0, t))),
        ),
        compiler_params=pltpu.CompilerParams(
            dimension_semantics=("parallel",),
            vmem_limit_bytes=vmem_limit),
        cost_estimate=pl.CostEstimate(
            flops=int(N * (2 * Dh * d_in * 2 + 3 * 2 * Dh * Dh + 2 * Dh)),
            transcendentals=int(N * 4 * Dh),
            bytes_accessed=int(4 * (d_in * N + Dh * num_tiles))),
    )(xT, W0c, b0c, W1c, b1c, W2c, W2x, b2c, W3c, b3c)

    mean = (jnp.sum(psum[:, :, 0], axis=0) * inv_n).reshape(Dh, 1)
    c4 = jnp.dot(W4b, mean) + b4c

    def pass1_kernel(x_ref, s1c_ref, W4a_ref, W4ax_ref, c4_ref, W5_ref,
                     W5x_ref, b5_ref, out_ref):
        xv = x_ref[...]
        s1 = s1c_ref[...].astype(f32)
        s = _sinpi(jnp.dot(W4a_ref[...], s1, preferred_element_type=f32)
                    + jnp.dot(W4ax_ref[...], xv, preferred_element_type=f32)
                    + c4_ref[...])
        out_ref[...] = (jnp.dot(W5_ref[...], s, preferred_element_type=f32)
                        + jnp.dot(W5x_ref[...], xv, preferred_element_type=f32)
                        + b5_ref[...])

    out = pl.pallas_call(
        pass1_kernel,
        out_shape=jax.ShapeDtypeStruct((d_out, N), f32),
        grid_spec=pltpu.PrefetchScalarGridSpec(
            num_scalar_prefetch=0,
            grid=(num_tiles,),
            in_specs=[
                pl.BlockSpec((d_in, tile_n), lambda t: (0, t)),
                pl.BlockSpec((Dh, tile_n), lambda t: (0, t)),
                pl.BlockSpec((Dp, Dh), lambda t: (0, 0)),
                pl.BlockSpec((Dp, d_in), lambda t: (0, 0)),
                pl.BlockSpec((Dp, 1), lambda t: (0, 0)),
                pl.BlockSpec((d_out, Dp), lambda t: (0, 0)),
                pl.BlockSpec((d_out, d_in), lambda t: (0, 0)),
                pl.BlockSpec((d_out, 1), lambda t: (0, 0)),
            ],
            out_specs=pl.BlockSpec((d_out, tile_n), lambda t: (0, t)),
        ),
        compiler_params=pltpu.CompilerParams(
            dimension_semantics=("parallel",),
            vmem_limit_bytes=vmem_limit),
        cost_estimate=pl.CostEstimate(
            flops=int(N * (2 * Dh * d_in * 2 + 2 * Dh * Dh + 2 * Dp * Dh
                           + 2 * d_out * Dp)),
            transcendentals=int(N * (2 * Dh + Dp)),
            bytes_accessed=int(4 * ((d_in + d_out) * N + Dh * num_tiles))),
    )(xT, s1c, W4a, W4ax, c4, W5c, W5x, b5c)

    return out.T


# speedup vs baseline: 1.3243x; 1.0931x over previous
import jax
import jax.numpy as jnp
from jax.experimental import pallas as pl
from jax.experimental.pallas import tpu as pltpu

_TILE_N = 32768


def _cdiv(a, b):
    return (a + b - 1) // b


def _sinpi(a):
    m = jnp.round(a)
    u = a - m
    u2 = u * u
    p = -0.554648779532642
    p = p * u2 + 2.541903899065775
    p = p * u2 - 5.167143330869833
    p = p * u2 + 3.1415820370344987
    su = u * p
    sb = (m.astype(jnp.int32) & 1) << 31
    return jax.lax.bitcast_convert_type(
        jax.lax.bitcast_convert_type(su, jnp.int32) ^ sb, jnp.float32)


def kernel(x, W0, b0, W1, b1, W2, b2, W3, b3, W4, b4, W5, b5):
    f32 = jnp.float32
    N, d_in = x.shape
    Dh = W0.shape[0]
    Dp = W4.shape[0]
    d_out = W5.shape[0]

    tile_n = _TILE_N
    num_tiles = _cdiv(N, tile_n)
    inv_n = 1.0 / N

    x = x.astype(f32)
    xT = x.T

    ip = 1.0 / jnp.pi
    W0c = W0.astype(f32) * ip
    b0c = b0.astype(f32).reshape(Dh, 1) * ip
    W1c = W1.astype(f32) * ip
    b1c = b1.astype(f32).reshape(Dh, 1) * ip
    W2c = W2.astype(f32) * ip
    W2x = W2c[:, :d_in]
    b2c = b2.astype(f32).reshape(Dh, 1) * ip
    W3c = W3.astype(f32) * ip
    b3c = b3.astype(f32).reshape(Dh, 1) * ip
    W4c = W4.astype(f32)
    W4a = W4c[:, :Dh] * ip
    W4ax = W4c[:, :d_in] * ip
    W4b = W4c[:, Dh:] * ip
    b4c = b4.astype(f32).reshape(Dp, 1) * ip
    W5c = W5.astype(f32)
    W5x = W5c[:, :d_in]
    b5c = b5.astype(f32).reshape(d_out, 1)

    vmem_limit = 48 * 1024 * 1024

    def pass0_kernel(x_ref, W0_ref, b0_ref, W1_ref, b1_ref, W2_ref, W2x_ref,
                     b2_ref, W3_ref, b3_ref, psum_ref, s1c_ref):
        t = pl.program_id(0)
        xv = x_ref[...]

        h = _sinpi(jnp.dot(W0_ref[...], xv, preferred_element_type=f32)
                    + b0_ref[...])
        s1 = _sinpi(jnp.dot(W1_ref[...], h, preferred_element_type=f32)
                     + b1_ref[...])
        s1c_ref[...] = s1.astype(jnp.bfloat16)
        u = _sinpi(jnp.dot(W2_ref[...], s1, preferred_element_type=f32)
                    + jnp.dot(W2x_ref[...], xv, preferred_element_type=f32)
                    + b2_ref[...])
        v = _sinpi(jnp.dot(W3_ref[...], u, preferred_element_type=f32)
                    + b3_ref[...])
        g = v + s1

        def emit(gv, xvv):
            ps = jnp.sum(gv, axis=1, keepdims=True)
            px = jnp.sum(xvv, axis=1, keepdims=True)
            psum_ref[...] = ps
            psum_ref[0:d_in, :] = ps[0:d_in, :] + px

        last_ragged = (t + 1) * tile_n > N

        @pl.when(jnp.logical_not(last_ragged))
        def _():
            emit(g, xv)

        @pl.when(last_ragged)
        def _():
            lane = jax.lax.broadcasted_iota(jnp.int32, (1, tile_n), 1)
            valid = (lane + t * tile_n) < N
            emit(jnp.where(valid, g, 0.0), jnp.where(valid, xv, 0.0))

    psum, s1c = pl.pallas_call(
        pass0_kernel,
        out_shape=(jax.ShapeDtypeStruct((num_tiles, Dh, 1), f32),
                   jax.ShapeDtypeStruct((Dh, N), jnp.bfloat16)),
        grid_spec=pltpu.PrefetchScalarGridSpec(
            num_scalar_prefetch=0,
            grid=(num_tiles,),
            in_specs=[
                pl.BlockSpec((d_in, tile_n), lambda t: (0, t)),
                pl.BlockSpec((Dh, d_in), lambda t: (0, 0)),
                pl.BlockSpec((Dh, 1), lambda t: (0, 0)),
                pl.BlockSpec((Dh, Dh), lambda t: (0, 0)),
                pl.BlockSpec((Dh, 1), lambda t: (0, 0)),
                pl.BlockSpec((Dh, Dh), lambda t: (0, 0)),
                pl.BlockSpec((Dh, d_in), lambda t: (0, 0)),
                pl.BlockSpec((Dh, 1), lambda t: (0, 0)),
                pl.BlockSpec((Dh, Dh), lambda t: (0, 0)),
                pl.BlockSpec((Dh, 1), lambda t: (0, 0)),
            ],
            out_specs=(pl.BlockSpec((None, Dh, 1), lambda t: (t, 0, 0)),
                       pl.BlockSpec((Dh, tile_n), lambda t: (0, t))),
        ),
        compiler_params=pltpu.CompilerParams(
            dimension_semantics=("parallel",),
            vmem_limit_bytes=vmem_limit),
        cost_estimate=pl.CostEstimate(
            flops=int(N * (2 * Dh * d_in * 2 + 3 * 2 * Dh * Dh + 2 * Dh)),
            transcendentals=int(N * 4 * Dh),
            bytes_accessed=int(4 * (d_in * N + Dh * num_tiles))),
    )(xT, W0c, b0c, W1c, b1c, W2c, W2x, b2c, W3c, b3c)

    mean = (jnp.sum(psum[:, :, 0], axis=0) * inv_n).reshape(Dh, 1)
    c4 = jnp.dot(W4b, mean) + b4c

    def pass1_kernel(x_ref, s1c_ref, W4a_ref, W4ax_ref, c4_ref, W5_ref,
                     W5x_ref, b5_ref, out_ref):
        xv = x_ref[...]
        s1 = s1c_ref[...].astype(f32)
        s = _sinpi(jnp.dot(W4a_ref[...], s1, preferred_element_type=f32)
                    + jnp.dot(W4ax_ref[...], xv, preferred_element_type=f32)
                    + c4_ref[...])
        out_ref[...] = (jnp.dot(W5_ref[...], s, preferred_element_type=f32)
                        + jnp.dot(W5x_ref[...], xv, preferred_element_type=f32)
                        + b5_ref[...])

    out = pl.pallas_call(
        pass1_kernel,
        out_shape=jax.ShapeDtypeStruct((d_out, N), f32),
        grid_spec=pltpu.PrefetchScalarGridSpec(
            num_scalar_prefetch=0,
            grid=(num_tiles,),
            in_specs=[
                pl.BlockSpec((d_in, tile_n), lambda t: (0, t)),
                pl.BlockSpec((Dh, tile_n), lambda t: (0, t)),
                pl.BlockSpec((Dp, Dh), lambda t: (0, 0)),
                pl.BlockSpec((Dp, d_in), lambda t: (0, 0)),
                pl.BlockSpec((Dp, 1), lambda t: (0, 0)),
                pl.BlockSpec((d_out, Dp), lambda t: (0, 0)),
                pl.BlockSpec((d_out, d_in), lambda t: (0, 0)),
                pl.BlockSpec((d_out, 1), lambda t: (0, 0)),
            ],
            out_specs=pl.BlockSpec((d_out, tile_n), lambda t: (0, t)),
        ),
        compiler_params=pltpu.CompilerParams(
            dimension_semantics=("parallel",),
            vmem_limit_bytes=vmem_limit),
        cost_estimate=pl.CostEstimate(
            flops=int(N * (2 * Dh * d_in * 2 + 2 * Dh * Dh + 2 * Dp * Dh
                           + 2 * d_out * Dp)),
            transcendentals=int(N * (2 * Dh + Dp)),
            bytes_accessed=int(4 * ((d_in + d_out) * N + Dh * num_tiles))),
    )(xT, s1c, W4a, W4ax, c4, W5c, W5x, b5c)

    return out.T


# speedup vs baseline: 1.3485x; 1.0182x over previous
import jax
import jax.numpy as jnp
from jax.experimental import pallas as pl
from jax.experimental.pallas import tpu as pltpu

_TILE_N = 32768


def _cdiv(a, b):
    return (a + b - 1) // b


def _sinpi(a):
    m = jnp.round(a)
    u = a - m
    u2 = u * u
    p = -0.554648779532642
    p = p * u2 + 2.541903899065775
    p = p * u2 - 5.167143330869833
    p = p * u2 + 3.1415820370344987
    su = u * p
    sb = (m.astype(jnp.int32) & 1) << 31
    return jax.lax.bitcast_convert_type(
        jax.lax.bitcast_convert_type(su, jnp.int32) ^ sb, jnp.float32)


def kernel(x, W0, b0, W1, b1, W2, b2, W3, b3, W4, b4, W5, b5):
    f32 = jnp.float32
    N, d_in = x.shape
    Dh = W0.shape[0]
    Dp = W4.shape[0]
    d_out = W5.shape[0]

    tile_n = _TILE_N
    num_tiles = _cdiv(N, tile_n)
    inv_n = 1.0 / N

    x = x.astype(f32)
    xT = x.T

    ip = 1.0 / jnp.pi
    W0c = W0.astype(f32) * ip
    b0c = b0.astype(f32).reshape(Dh, 1) * ip
    W0a = jnp.concatenate([W0c, b0c], axis=1)
    W1c = W1.astype(f32) * ip
    b1c = b1.astype(f32).reshape(Dh, 1) * ip
    W2c = W2.astype(f32) * ip
    b2c = b2.astype(f32).reshape(Dh, 1) * ip
    W2xa = jnp.concatenate([W2c[:, :d_in], b2c], axis=1)
    W3c = W3.astype(f32) * ip
    b3c = b3.astype(f32).reshape(Dh, 1) * ip
    W4c = W4.astype(f32)
    W4a = W4c[:, :Dh] * ip
    W4ax = W4c[:, :d_in] * ip
    W4b = W4c[:, Dh:] * ip
    b4c = b4.astype(f32).reshape(Dp, 1) * ip
    W5c = W5.astype(f32)
    W5x = W5c[:, :d_in]
    b5c = b5.astype(f32).reshape(d_out, 1)

    vmem_limit = 48 * 1024 * 1024

    def pass0_kernel(x_ref, W0a_ref, W1_ref, b1_ref, W2_ref, W2xa_ref,
                     W3_ref, b3_ref, psum_ref, s1c_ref, xa_ref):
        t = pl.program_id(0)
        xv = x_ref[...]
        xa_ref[0:d_in, :] = xv
        xa_ref[d_in:d_in + 1, :] = jnp.full((1, tile_n), 1.0, f32)
        xa = xa_ref[...]

        h = _sinpi(jnp.dot(W0a_ref[...], xa, preferred_element_type=f32))
        s1 = _sinpi(jnp.dot(W1_ref[...], h, preferred_element_type=f32)
                     + b1_ref[...])
        s1c_ref[...] = s1.astype(jnp.bfloat16)
        u = _sinpi(jnp.dot(W2_ref[...], s1, preferred_element_type=f32)
                    + jnp.dot(W2xa_ref[...], xa, preferred_element_type=f32))
        v = _sinpi(jnp.dot(W3_ref[...], u, preferred_element_type=f32)
                    + b3_ref[...])
        g = v + s1

        def emit(gv, xvv):
            ps = jnp.sum(gv, axis=1, keepdims=True)
            px = jnp.sum(xvv, axis=1, keepdims=True)
            psum_ref[...] = ps
            psum_ref[0:d_in, :] = ps[0:d_in, :] + px

        last_ragged = (t + 1) * tile_n > N

        @pl.when(jnp.logical_not(last_ragged))
        def _():
            emit(g, xv)

        @pl.when(last_ragged)
        def _():
            lane = jax.lax.broadcasted_iota(jnp.int32, (1, tile_n), 1)
            valid = (lane + t * tile_n) < N
            emit(jnp.where(valid, g, 0.0), jnp.where(valid, xv, 0.0))

    psum, s1c = pl.pallas_call(
        pass0_kernel,
        out_shape=(jax.ShapeDtypeStruct((num_tiles, Dh, 1), f32),
                   jax.ShapeDtypeStruct((Dh, N), jnp.bfloat16)),
        grid_spec=pltpu.PrefetchScalarGridSpec(
            num_scalar_prefetch=0,
            grid=(num_tiles,),
            in_specs=[
                pl.BlockSpec((d_in, tile_n), lambda t: (0, t)),
                pl.BlockSpec((Dh, d_in + 1), lambda t: (0, 0)),
                pl.BlockSpec((Dh, Dh), lambda t: (0, 0)),
                pl.BlockSpec((Dh, 1), lambda t: (0, 0)),
                pl.BlockSpec((Dh, Dh), lambda t: (0, 0)),
                pl.BlockSpec((Dh, d_in + 1), lambda t: (0, 0)),
                pl.BlockSpec((Dh, Dh), lambda t: (0, 0)),
                pl.BlockSpec((Dh, 1), lambda t: (0, 0)),
            ],
            out_specs=(pl.BlockSpec((None, Dh, 1), lambda t: (t, 0, 0)),
                       pl.BlockSpec((Dh, tile_n), lambda t: (0, t))),
            scratch_shapes=[pltpu.VMEM((d_in + 1, tile_n), f32)],
        ),
        compiler_params=pltpu.CompilerParams(
            dimension_semantics=("parallel",),
            vmem_limit_bytes=vmem_limit),
        cost_estimate=pl.CostEstimate(
            flops=int(N * (2 * Dh * d_in * 2 + 3 * 2 * Dh * Dh + 2 * Dh)),
            transcendentals=int(N * 4 * Dh),
            bytes_accessed=int(4 * (d_in * N + Dh * num_tiles))),
    )(xT, W0a, W1c, b1c, W2c, W2xa, W3c, b3c)

    mean = (jnp.sum(psum[:, :, 0], axis=0) * inv_n).reshape(Dh, 1)
    c4 = jnp.dot(W4b, mean) + b4c
    W4axc = jnp.concatenate([W4ax, c4], axis=1)
    W5xa = jnp.concatenate([W5x, b5c], axis=1)

    def pass1_kernel(x_ref, s1c_ref, W4a_ref, W4axc_ref, W5_ref,
                     W5xa_ref, out_ref, xa_ref):
        xa_ref[0:d_in, :] = x_ref[...]
        xa_ref[d_in:d_in + 1, :] = jnp.full((1, tile_n), 1.0, f32)
        xa = xa_ref[...]
        s1 = s1c_ref[...].astype(f32)
        s = _sinpi(jnp.dot(W4a_ref[...], s1, preferred_element_type=f32)
                    + jnp.dot(W4axc_ref[...], xa, preferred_element_type=f32))
        out_ref[...] = (jnp.dot(W5_ref[...], s, preferred_element_type=f32)
                        + jnp.dot(W5xa_ref[...], xa, preferred_element_type=f32))

    out = pl.pallas_call(
        pass1_kernel,
        out_shape=jax.ShapeDtypeStruct((d_out, N), f32),
        grid_spec=pltpu.PrefetchScalarGridSpec(
            num_scalar_prefetch=0,
            grid=(num_tiles,),
            in_specs=[
                pl.BlockSpec((d_in, tile_n), lambda t: (0, t)),
                pl.BlockSpec((Dh, tile_n), lambda t: (0, t)),
                pl.BlockSpec((Dp, Dh), lambda t: (0, 0)),
                pl.BlockSpec((Dp, d_in + 1), lambda t: (0, 0)),
                pl.BlockSpec((d_out, Dp), lambda t: (0, 0)),
                pl.BlockSpec((d_out, d_in + 1), lambda t: (0, 0)),
            ],
            out_specs=pl.BlockSpec((d_out, tile_n), lambda t: (0, t)),
            scratch_shapes=[pltpu.VMEM((d_in + 1, tile_n), f32)],
        ),
        compiler_params=pltpu.CompilerParams(
            dimension_semantics=("parallel",),
            vmem_limit_bytes=vmem_limit),
        cost_estimate=pl.CostEstimate(
            flops=int(N * (2 * Dh * d_in * 2 + 2 * Dh * Dh + 2 * Dp * Dh
                           + 2 * d_out * Dp)),
            transcendentals=int(N * (2 * Dh + Dp)),
            bytes_accessed=int(4 * ((d_in + d_out) * N + Dh * num_tiles))),
    )(xT, s1c, W4a, W4axc, W5c, W5xa)

    return out.T


# speedup vs baseline: 1.4809x; 1.0982x over previous
import jax
import jax.numpy as jnp
from jax.experimental import pallas as pl
from jax.experimental.pallas import tpu as pltpu

_TILE_N = 32768


def _cdiv(a, b):
    return (a + b - 1) // b


def _sinpi(a):
    m = jnp.round(a)
    u = a - m
    u2 = u * u
    p = 2.299631306230256
    p = p * u2 - 5.136926968822117
    p = p * u2 + 3.1406411135604313
    su = u * p
    sb = (m.astype(jnp.int32) & 1) << 31
    return jax.lax.bitcast_convert_type(
        jax.lax.bitcast_convert_type(su, jnp.int32) ^ sb, jnp.float32)


def kernel(x, W0, b0, W1, b1, W2, b2, W3, b3, W4, b4, W5, b5):
    f32 = jnp.float32
    N, d_in = x.shape
    Dh = W0.shape[0]
    Dp = W4.shape[0]
    d_out = W5.shape[0]

    tile_n = _TILE_N
    num_tiles = _cdiv(N, tile_n)
    inv_n = 1.0 / N

    x = x.astype(f32)
    xT = x.T

    ip = 1.0 / jnp.pi
    W0c = W0.astype(f32) * ip
    b0c = b0.astype(f32).reshape(Dh, 1) * ip
    W0a = jnp.concatenate([W0c, b0c], axis=1)
    W1c = W1.astype(f32) * ip
    b1c = b1.astype(f32).reshape(Dh, 1) * ip
    W2c = W2.astype(f32) * ip
    b2c = b2.astype(f32).reshape(Dh, 1) * ip
    W2xa = jnp.concatenate([W2c[:, :d_in], b2c], axis=1)
    W3c = W3.astype(f32) * ip
    b3c = b3.astype(f32).reshape(Dh, 1) * ip
    W4c = W4.astype(f32)
    W4a = W4c[:, :Dh] * ip
    W4ax = W4c[:, :d_in] * ip
    W4b = W4c[:, Dh:] * ip
    b4c = b4.astype(f32).reshape(Dp, 1) * ip
    W5c = W5.astype(f32)
    W5x = W5c[:, :d_in]
    b5c = b5.astype(f32).reshape(d_out, 1)

    vmem_limit = 48 * 1024 * 1024

    def pass0_kernel(x_ref, W0a_ref, W1_ref, b1_ref, W2_ref, W2xa_ref,
                     W3_ref, b3_ref, psum_ref, s1c_ref, xa_ref):
        t = pl.program_id(0)
        xv = x_ref[...]
        xa_ref[0:d_in, :] = xv
        xa_ref[d_in:d_in + 1, :] = jnp.full((1, tile_n), 1.0, f32)
        xa = xa_ref[...]

        h = _sinpi(jnp.dot(W0a_ref[...], xa, preferred_element_type=f32))
        s1 = _sinpi(jnp.dot(W1_ref[...], h, preferred_element_type=f32)
                     + b1_ref[...])
        s1c_ref[...] = s1.astype(jnp.bfloat16)
        u = _sinpi(jnp.dot(W2_ref[...], s1, preferred_element_type=f32)
                    + jnp.dot(W2xa_ref[...], xa, preferred_element_type=f32))
        v = _sinpi(jnp.dot(W3_ref[...], u, preferred_element_type=f32)
                    + b3_ref[...])
        g = v + s1

        def emit(gv, xvv):
            ps = jnp.sum(gv, axis=1, keepdims=True)
            px = jnp.sum(xvv, axis=1, keepdims=True)
            psum_ref[...] = ps
            psum_ref[0:d_in, :] = ps[0:d_in, :] + px

        last_ragged = (t + 1) * tile_n > N

        @pl.when(jnp.logical_not(last_ragged))
        def _():
            emit(g, xv)

        @pl.when(last_ragged)
        def _():
            lane = jax.lax.broadcasted_iota(jnp.int32, (1, tile_n), 1)
            valid = (lane + t * tile_n) < N
            emit(jnp.where(valid, g, 0.0), jnp.where(valid, xv, 0.0))

    psum, s1c = pl.pallas_call(
        pass0_kernel,
        out_shape=(jax.ShapeDtypeStruct((num_tiles, Dh, 1), f32),
                   jax.ShapeDtypeStruct((Dh, N), jnp.bfloat16)),
        grid_spec=pltpu.PrefetchScalarGridSpec(
            num_scalar_prefetch=0,
            grid=(num_tiles,),
            in_specs=[
                pl.BlockSpec((d_in, tile_n), lambda t: (0, t)),
                pl.BlockSpec((Dh, d_in + 1), lambda t: (0, 0)),
                pl.BlockSpec((Dh, Dh), lambda t: (0, 0)),
                pl.BlockSpec((Dh, 1), lambda t: (0, 0)),
                pl.BlockSpec((Dh, Dh), lambda t: (0, 0)),
                pl.BlockSpec((Dh, d_in + 1), lambda t: (0, 0)),
                pl.BlockSpec((Dh, Dh), lambda t: (0, 0)),
                pl.BlockSpec((Dh, 1), lambda t: (0, 0)),
            ],
            out_specs=(pl.BlockSpec((None, Dh, 1), lambda t: (t, 0, 0)),
                       pl.BlockSpec((Dh, tile_n), lambda t: (0, t))),
            scratch_shapes=[pltpu.VMEM((d_in + 1, tile_n), f32)],
        ),
        compiler_params=pltpu.CompilerParams(
            dimension_semantics=("parallel",),
            vmem_limit_bytes=vmem_limit),
        cost_estimate=pl.CostEstimate(
            flops=int(N * (2 * Dh * d_in * 2 + 3 * 2 * Dh * Dh + 2 * Dh)),
            transcendentals=int(N * 4 * Dh),
            bytes_accessed=int(4 * (d_in * N + Dh * num_tiles))),
    )(xT, W0a, W1c, b1c, W2c, W2xa, W3c, b3c)

    mean = (jnp.sum(psum[:, :, 0], axis=0) * inv_n).reshape(Dh, 1)
    c4 = jnp.dot(W4b, mean) + b4c
    W4axc = jnp.concatenate([W4ax, c4], axis=1)
    W5xa = jnp.concatenate([W5x, b5c], axis=1)

    def pass1_kernel(x_ref, s1c_ref, W4a_ref, W4axc_ref, W5_ref,
                     W5xa_ref, out_ref, xa_ref):
        xa_ref[0:d_in, :] = x_ref[...]
        xa_ref[d_in:d_in + 1, :] = jnp.full((1, tile_n), 1.0, f32)
        xa = xa_ref[...]
        s1 = s1c_ref[...].astype(f32)
        s = _sinpi(jnp.dot(W4a_ref[...], s1, preferred_element_type=f32)
                    + jnp.dot(W4axc_ref[...], xa, preferred_element_type=f32))
        out_ref[...] = (jnp.dot(W5_ref[...], s, preferred_element_type=f32)
                        + jnp.dot(W5xa_ref[...], xa, preferred_element_type=f32))

    out = pl.pallas_call(
        pass1_kernel,
        out_shape=jax.ShapeDtypeStruct((d_out, N), f32),
        grid_spec=pltpu.PrefetchScalarGridSpec(
            num_scalar_prefetch=0,
            grid=(num_tiles,),
            in_specs=[
                pl.BlockSpec((d_in, tile_n), lambda t: (0, t)),
                pl.BlockSpec((Dh, tile_n), lambda t: (0, t)),
                pl.BlockSpec((Dp, Dh), lambda t: (0, 0)),
                pl.BlockSpec((Dp, d_in + 1), lambda t: (0, 0)),
                pl.BlockSpec((d_out, Dp), lambda t: (0, 0)),
                pl.BlockSpec((d_out, d_in + 1), lambda t: (0, 0)),
            ],
            out_specs=pl.BlockSpec((d_out, tile_n), lambda t: (0, t)),
            scratch_shapes=[pltpu.VMEM((d_in + 1, tile_n), f32)],
        ),
        compiler_params=pltpu.CompilerParams(
            dimension_semantics=("parallel",),
            vmem_limit_bytes=vmem_limit),
        cost_estimate=pl.CostEstimate(
            flops=int(N * (2 * Dh * d_in * 2 + 2 * Dh * Dh + 2 * Dp * Dh
                           + 2 * d_out * Dp)),
            transcendentals=int(N * (2 * Dh + Dp)),
            bytes_accessed=int(4 * ((d_in + d_out) * N + Dh * num_tiles))),
    )(xT, s1c, W4a, W4axc, W5c, W5xa)

    return out.T
